# baseline (speedup 1.0000x reference)
import functools

import jax
import jax.numpy as jnp
from jax import lax
from jax.experimental import pallas as pl
from jax.experimental.pallas import tpu as pltpu
from jax.experimental.pallas import tpu_sc as plsc

EMB = 128
VOCAB = 27
PEP = 31
PEPP = 32
BATCH = 16384
NC, NS = 2, 16
NW = NC * NS
BPW = BATCH // NW
KB = 4
KROW = KB * PEPP
NCHUNK = BPW // KB
NBUF = 4
OROWS = BATCH * PEPP


def _prep_body(x_ref, aa_ref, pos_ref, comb_ref, idx_ref):
    comb_ref[...] = aa_ref[...][:, None, :] + pos_ref[...][None, :, :]
    p = lax.broadcasted_iota(jnp.int32, (BATCH, PEP), 1)
    idx_ref[...] = jnp.concatenate(
        [x_ref[...] * PEP + p, jnp.zeros((BATCH, 1), jnp.int32)], axis=1)


def _prep(x, aa_table, pos_table):
    return pl.pallas_call(
        _prep_body,
        out_shape=(
            jax.ShapeDtypeStruct((VOCAB, PEP, EMB), jnp.float32),
            jax.ShapeDtypeStruct((BATCH, PEPP), jnp.int32),
        ),
    )(x, aa_table, pos_table)


def _sc_gather(comb, idx3):
    mesh = plsc.VectorSubcoreMesh(core_axis_name="c", subcore_axis_name="s")

    @functools.partial(
        pl.kernel,
        mesh=mesh,
        out_type=jax.ShapeDtypeStruct((OROWS, EMB), jnp.float32),
        scratch_types=[
            pltpu.VMEM((NCHUNK, KROW), jnp.int32),
            *[pltpu.VMEM((KROW, EMB), jnp.float32) for _ in range(NBUF)],
            pltpu.SemaphoreType.DMA((NBUF,)),
            pltpu.SemaphoreType.DMA((NBUF,)),
        ],
    )
    def k(comb_hbm, idx_hbm, out_hbm, idx_all, r0, r1, r2, r3, gsem, osem):
        rows = [r0, r1, r2, r3]
        wid = lax.axis_index("s") * NC + lax.axis_index("c")
        base = wid * NCHUNK * KROW
        pltpu.sync_copy(idx_hbm.at[wid], idx_all)

        def wait_gather(s):
            pltpu.make_async_copy(
                comb_hbm.at[idx_all.at[0]], rows[s], gsem.at[s]).wait()

        def wait_out(s):
            pltpu.make_async_copy(
                rows[s], out_hbm.at[pl.ds(base, KROW)], osem.at[s]).wait()

        def start_gather(j, s):
            pltpu.async_copy(
                comb_hbm.at[idx_all.at[j]], rows[s], gsem.at[s])

        def start_out(j, s):
            pltpu.async_copy(
                rows[s], out_hbm.at[pl.ds(base + j * KROW, KROW)],
                osem.at[s])

        def body(g, _):
            for s in range(NBUF):
                j = g * NBUF + s
                pl.when(g > 0)(lambda s=s: wait_out(s))
                start_gather(j, s)
                ps = (s - 1) % NBUF
                if s == 0:
                    def prev(g=g, ps=ps):
                        wait_gather(ps)
                        start_out(g * NBUF - 1, ps)
                    pl.when(g > 0)(prev)
                else:
                    wait_gather(ps)
                    start_out(j - 1, ps)
            return 0

        lax.fori_loop(0, NCHUNK // NBUF, body, 0)
        wait_gather(NBUF - 1)
        start_out(NCHUNK - 1, NBUF - 1)
        for s in range(NBUF):
            wait_out(s)

    return k(comb, idx3)


def kernel(x, aa_table, pos_table):
    x32 = x.astype(jnp.int32)
    comb3, idx = _prep(x32, aa_table, pos_table)
    comb = comb3.reshape(VOCAB * PEP, EMB)
    out = _sc_gather(comb, idx.reshape(NW, NCHUNK, KROW))
    return out.reshape(BATCH, PEPP, EMB)[:, :PEP, :]
